# baseline (speedup 1.0000x reference)
import functools

import jax
import jax.numpy as jnp
from jax import lax
from jax.experimental import pallas as pl
from jax.experimental.pallas import tpu as pltpu
from jax.experimental.pallas import tpu_sc as plsc

B, L = 16384, 16
V, D = 100000, 768

NC, NS = 2, 16
NW = NC * NS
B_PER_W = B // NW
CHUNK = 64
SIZES = (16, 32) + (64,) * 7 + (16,)
OFFS = tuple(sum(SIZES[:j]) for j in range(len(SIZES)))
NCHUNK = len(SIZES)
NBUF = 2
INFLIGHT = 2

_mesh = plsc.VectorSubcoreMesh(core_axis_name="c", subcore_axis_name="s")


@functools.partial(
    pl.kernel,
    mesh=_mesh,
    compiler_params=pltpu.CompilerParams(needs_layout_passes=False),
    out_type=(
        jax.ShapeDtypeStruct((B, D), jnp.float32),
        jax.ShapeDtypeStruct((B,), jnp.float32),
    ),
    scratch_types=[
        pltpu.VMEM((B_PER_W,), jnp.int32),
        pltpu.VMEM((NBUF, CHUNK, D), jnp.float32),
        pltpu.VMEM((B_PER_W,), jnp.float32),
        pltpu.VMEM_SHARED((V,), jnp.float32),
        [pltpu.SemaphoreType.DMA] * NBUF,
        [pltpu.SemaphoreType.DMA] * NBUF,
        pltpu.SemaphoreType.DMA,
    ],
)
def _gather_kernel(ids_hbm, table_hbm, bias_hbm, out_hbm,
                   bias_out_hbm, idx_v, rows_v, bias_v, bias_sp,
                   gsems, wsems, semb):
    sid = lax.axis_index("s")
    wid = sid * NC + lax.axis_index("c")
    base = wid * B_PER_W

    pltpu.sync_copy(ids_hbm.at[pl.ds(base, B_PER_W)], idx_v)

    def start_gather(j):
        return pltpu.async_copy(
            table_hbm.at[idx_v.at[pl.ds(OFFS[j], SIZES[j])]],
            rows_v.at[j % NBUF, pl.ds(0, SIZES[j])], gsems[j % NBUF])

    def start_write(j):
        return pltpu.async_copy(
            rows_v.at[j % NBUF, pl.ds(0, SIZES[j])],
            out_hbm.at[pl.ds(base + OFFS[j], SIZES[j])], wsems[j % NBUF])

    g = {j: start_gather(j) for j in range(INFLIGHT)}

    @pl.when(sid == 0)
    def _():
        pltpu.sync_copy(bias_hbm, bias_sp)
    plsc.subcore_barrier()

    bias_copies = [
        pltpu.async_copy(bias_sp.at[idx_v.at[pl.ds(q * 128, 128)]],
                         bias_v.at[pl.ds(q * 128, 128)], semb)
        for q in range(B_PER_W // 128)
    ]

    w = {}
    for j in range(NCHUNK):
        g[j].wait()
        w[j] = start_write(j)
        k = j + INFLIGHT
        if k < NCHUNK:
            if k - NBUF >= 0:
                w[k - NBUF].wait()
            g[k] = start_gather(k)
    waited = {k - NBUF for k in range(INFLIGHT, NCHUNK) if k - NBUF >= 0}
    for j in range(NCHUNK):
        if j not in waited:
            w[j].wait()

    for c in bias_copies:
        c.wait()
    pltpu.sync_copy(bias_v, bias_out_hbm.at[pl.ds(base, B_PER_W)])


def kernel(target_surface_forms, target_priors, input_embeddings, bias):
    del target_priors
    ids = target_surface_forms[:, 0].astype(jnp.int32)
    emb, b = _gather_kernel(ids, input_embeddings, bias.reshape(V))
    return emb, b


# speedup vs baseline: 1.0074x; 1.0074x over previous
import functools

import jax
import jax.numpy as jnp
from jax import lax
from jax.experimental import pallas as pl
from jax.experimental.pallas import tpu as pltpu
from jax.experimental.pallas import tpu_sc as plsc

B, L = 16384, 16
V, D = 100000, 768

NC, NS = 2, 16
NW = NC * NS
B_PER_W = B // NW
CHUNK = 64
SIZES = (32,) + (64,) * 7 + (32,)
OFFS = tuple(sum(SIZES[:j]) for j in range(len(SIZES)))
NCHUNK = len(SIZES)
NBUF = 2
INFLIGHT = 2

_mesh = plsc.VectorSubcoreMesh(core_axis_name="c", subcore_axis_name="s")


@functools.partial(
    pl.kernel,
    mesh=_mesh,
    compiler_params=pltpu.CompilerParams(needs_layout_passes=False),
    out_type=(
        jax.ShapeDtypeStruct((B, D), jnp.float32),
        jax.ShapeDtypeStruct((B,), jnp.float32),
    ),
    scratch_types=[
        pltpu.VMEM((B_PER_W,), jnp.int32),
        pltpu.VMEM((NBUF, CHUNK, D), jnp.float32),
        pltpu.VMEM((B_PER_W,), jnp.float32),
        pltpu.VMEM_SHARED((V,), jnp.float32),
        [pltpu.SemaphoreType.DMA] * NBUF,
        [pltpu.SemaphoreType.DMA] * NBUF,
        pltpu.SemaphoreType.DMA,
    ],
)
def _gather_kernel(ids_hbm, table_hbm, bias_hbm, out_hbm,
                   bias_out_hbm, idx_v, rows_v, bias_v, bias_sp,
                   gsems, wsems, semb):
    sid = lax.axis_index("s")
    wid = sid * NC + lax.axis_index("c")
    base = wid * B_PER_W

    pltpu.sync_copy(ids_hbm.at[pl.ds(base, B_PER_W)], idx_v)

    def start_gather(j):
        return pltpu.async_copy(
            table_hbm.at[idx_v.at[pl.ds(OFFS[j], SIZES[j])]],
            rows_v.at[j % NBUF, pl.ds(0, SIZES[j])], gsems[j % NBUF])

    def start_write(j):
        return pltpu.async_copy(
            rows_v.at[j % NBUF, pl.ds(0, SIZES[j])],
            out_hbm.at[pl.ds(base + OFFS[j], SIZES[j])], wsems[j % NBUF])

    g = {j: start_gather(j) for j in range(INFLIGHT)}

    @pl.when(sid == 0)
    def _():
        pltpu.sync_copy(bias_hbm, bias_sp)
    plsc.subcore_barrier()

    bias_copies = [
        pltpu.async_copy(bias_sp.at[idx_v.at[pl.ds(q * 128, 128)]],
                         bias_v.at[pl.ds(q * 128, 128)], semb)
        for q in range(B_PER_W // 128)
    ]

    w = {}
    for j in range(NCHUNK):
        g[j].wait()
        w[j] = start_write(j)
        k = j + INFLIGHT
        if k < NCHUNK:
            if k - NBUF >= 0:
                w[k - NBUF].wait()
            g[k] = start_gather(k)
    waited = {k - NBUF for k in range(INFLIGHT, NCHUNK) if k - NBUF >= 0}
    for j in range(NCHUNK):
        if j not in waited:
            w[j].wait()

    for c in bias_copies:
        c.wait()
    pltpu.sync_copy(bias_v, bias_out_hbm.at[pl.ds(base, B_PER_W)])


def kernel(target_surface_forms, target_priors, input_embeddings, bias):
    del target_priors
    ids = target_surface_forms[:, 0].astype(jnp.int32)
    emb, b = _gather_kernel(ids, input_embeddings, bias.reshape(V))
    return emb, b
